# Initial kernel scaffold; baseline (speedup 1.0000x reference)
import jax
import jax.numpy as jnp
from jax import lax
from jax.experimental import pallas as pl
from jax.experimental.pallas import tpu as pltpu
from jax.experimental.pallas import tpu_sc as plsc

N = 10000
E = 320000
D = 128
D_CAT = 32
D_CONT = 96
G = 256

NC = 2
NS = 16
NW = NC * NS
EPW = E // NW
K = 80
CH = EPW // K
RPT = N // NS
DW = 16
ZR = 125

R = 2000
NG = N // R

_HI = jax.lax.Precision.HIGHEST
_mesh = plsc.VectorSubcoreMesh(core_axis_name="c", subcore_axis_name="s")


def _deg_body(col_hbm, out_hbm, colb, onesb, zb, acc, sem):
    cid = lax.axis_index("c")
    sid = lax.axis_index("s")
    wid = sid * NC + cid

    def _ofill(i, _):
        onesb[i, :] = jnp.full((DW,), 1.0, jnp.float32)
        return 0

    lax.fori_loop(0, K, _ofill, 0)

    def _zfill(i, _):
        zb[i, :] = jnp.zeros((DW,), jnp.float32)
        return 0

    lax.fori_loop(0, RPT, _zfill, 0)
    pltpu.sync_copy(zb, acc.at[pl.ds(sid * RPT, RPT)])
    plsc.subcore_barrier()

    base = wid * EPW

    def _step(j, _):
        off = pl.multiple_of(base + j * K, 8)
        pltpu.sync_copy(col_hbm.at[pl.ds(off, K)], colb)
        pltpu.sync_copy(onesb, acc.at[colb], add=True)
        return 0

    lax.fori_loop(0, CH, _step, 0)
    plsc.subcore_barrier()
    r0 = sid * RPT
    pltpu.sync_copy(acc.at[pl.ds(r0, RPT)], out_hbm.at[cid].at[pl.ds(r0, RPT)])


_deg_call = pl.kernel(
    _deg_body,
    out_type=jax.ShapeDtypeStruct((NC, N, DW), jnp.float32),
    mesh=_mesh,
    scratch_types=[
        pltpu.VMEM((K,), jnp.int32),
        pltpu.VMEM((K, DW), jnp.float32),
        pltpu.VMEM((RPT, DW), jnp.float32),
        pltpu.VMEM_SHARED((N, DW), jnp.float32),
        pltpu.SemaphoreType.DMA,
    ],
)


def _conv_body(xs_hbm, row_hbm, col_hbm, out_hbm, rowb, colb, gbuf, zb, acc, sem):
    cid = lax.axis_index("c")
    sid = lax.axis_index("s")
    wid = sid * NC + cid

    def _zfill(i, _):
        for j in range(D // 16):
            zb[i, pl.ds(j * 16, 16)] = jnp.zeros((16,), jnp.float32)
        return 0

    lax.fori_loop(0, ZR, _zfill, 0)
    for p in range(RPT // ZR):
        pltpu.sync_copy(zb, acc.at[pl.ds(sid * RPT + p * ZR, ZR)])
    plsc.subcore_barrier()

    base = wid * EPW

    def _step(j, _):
        off = pl.multiple_of(base + j * K, 8)
        pltpu.sync_copy(row_hbm.at[pl.ds(off, K)], rowb)
        pltpu.async_copy(xs_hbm.at[rowb], gbuf, sem).wait()
        pltpu.sync_copy(col_hbm.at[pl.ds(off, K)], colb)
        pltpu.sync_copy(gbuf, acc.at[colb], add=True)
        return 0

    lax.fori_loop(0, CH, _step, 0)
    plsc.subcore_barrier()
    r0 = sid * RPT
    pltpu.sync_copy(acc.at[pl.ds(r0, RPT)], out_hbm.at[cid].at[pl.ds(r0, RPT)])


_conv_call = pl.kernel(
    _conv_body,
    out_type=jax.ShapeDtypeStruct((NC, N, D), jnp.float32),
    mesh=_mesh,
    scratch_types=[
        pltpu.VMEM((K,), jnp.int32),
        pltpu.VMEM((K,), jnp.int32),
        pltpu.VMEM((K, D), jnp.float32),
        pltpu.VMEM((ZR, D), jnp.float32),
        pltpu.VMEM_SHARED((N, D), jnp.float32),
        pltpu.SemaphoreType.DMA,
    ],
)


def _elu(v):
    return jnp.where(v > 0, v, jnp.exp(jnp.minimum(v, 0.0)) - 1.0)


def _ab_body(x_ref, degs_ref, wcat_ref, bcat_ref, wi_ref, bi_ref, w1_ref,
             xs1_ref, dis_ref):
    xb = x_ref[...]
    degs = degs_ref[...]
    deg = (degs[0, :] + degs[1, :] + 1.0)[:, None]
    dis = lax.rsqrt(deg)
    s = jnp.dot(xb[:, :D_CAT], wcat_ref[...], preferred_element_type=jnp.float32,
                precision=_HI) + bcat_ref[...]
    s = jnp.maximum(s, 0.0)
    h = (jnp.dot(xb[:, D_CAT:], wi_ref[...][:D_CONT],
                 preferred_element_type=jnp.float32, precision=_HI)
         + jnp.dot(s, wi_ref[...][D_CONT:],
                   preferred_element_type=jnp.float32, precision=_HI)
         + bi_ref[...])
    h = jnp.maximum(h, 0.0)
    xs1_ref[...] = jnp.dot(h, w1_ref[...], preferred_element_type=jnp.float32,
                           precision=_HI) * dis
    dis_ref[...] = dis


_ab_call = pl.pallas_call(
    _ab_body,
    grid=(NG,),
    in_specs=[
        pl.BlockSpec((R, D), lambda i: (i, 0)),
        pl.BlockSpec((NC, R), lambda i: (0, i)),
        pl.BlockSpec((D_CAT, D_CAT), lambda i: (0, 0)),
        pl.BlockSpec((1, D_CAT), lambda i: (0, 0)),
        pl.BlockSpec((D, D), lambda i: (0, 0)),
        pl.BlockSpec((1, D), lambda i: (0, 0)),
        pl.BlockSpec((D, D), lambda i: (0, 0)),
    ],
    out_specs=[
        pl.BlockSpec((R, D), lambda i: (i, 0)),
        pl.BlockSpec((R, 1), lambda i: (i, 0)),
    ],
    out_shape=[
        jax.ShapeDtypeStruct((N, D), jnp.float32),
        jax.ShapeDtypeStruct((N, 1), jnp.float32),
    ],
)


def _c_body(acc_ref, xs_ref, dis_ref, b_ref, w_ref, out_ref):
    a = acc_ref[0] + acc_ref[1] + xs_ref[...]
    dis = dis_ref[...]
    h = _elu(a * dis + b_ref[...])
    out_ref[...] = jnp.dot(h, w_ref[...], preferred_element_type=jnp.float32,
                           precision=_HI) * dis


_c_call = pl.pallas_call(
    _c_body,
    grid=(NG,),
    in_specs=[
        pl.BlockSpec((NC, R, D), lambda i: (0, i, 0)),
        pl.BlockSpec((R, D), lambda i: (i, 0)),
        pl.BlockSpec((R, 1), lambda i: (i, 0)),
        pl.BlockSpec((1, D), lambda i: (0, 0)),
        pl.BlockSpec((D, D), lambda i: (0, 0)),
    ],
    out_specs=pl.BlockSpec((R, D), lambda i: (i, 0)),
    out_shape=jax.ShapeDtypeStruct((N, D), jnp.float32),
)


def _d_body(acc_ref, xs_ref, dis_ref, b_ref, batch_ref, wout_ref, bout_ref,
            out_ref, pooled_acc, cnt_acc):
    i = pl.program_id(0)

    @pl.when(i == 0)
    def _():
        pooled_acc[...] = jnp.zeros_like(pooled_acc)
        cnt_acc[...] = jnp.zeros_like(cnt_acc)

    a = acc_ref[0] + acc_ref[1] + xs_ref[...]
    dis = dis_ref[...]
    h = _elu(a * dis + b_ref[...])
    bb = batch_ref[0, 0, :]
    oh = (bb[:, None] == lax.broadcasted_iota(jnp.int32, (R, G), 1)
          ).astype(jnp.float32)
    pooled_acc[...] += lax.dot_general(
        oh, h, (((0,), (0,)), ((), ())), preferred_element_type=jnp.float32,
        precision=_HI)
    cnt_acc[...] += jnp.sum(oh, axis=0)[:, None]

    @pl.when(i == pl.num_programs(0) - 1)
    def _():
        pooled = pooled_acc[...] / jnp.maximum(cnt_acc[...], 1.0)
        out_ref[...] = jnp.dot(pooled, wout_ref[...],
                               preferred_element_type=jnp.float32,
                               precision=_HI) + bout_ref[...]


_d_call = pl.pallas_call(
    _d_body,
    grid=(NG,),
    in_specs=[
        pl.BlockSpec((NC, R, D), lambda i: (0, i, 0)),
        pl.BlockSpec((R, D), lambda i: (i, 0)),
        pl.BlockSpec((R, 1), lambda i: (i, 0)),
        pl.BlockSpec((1, D), lambda i: (0, 0)),
        pl.BlockSpec((1, 1, R), lambda i: (i, 0, 0)),
        pl.BlockSpec((D, D), lambda i: (0, 0)),
        pl.BlockSpec((1, D), lambda i: (0, 0)),
    ],
    out_specs=pl.BlockSpec((G, D), lambda i: (0, 0)),
    out_shape=jax.ShapeDtypeStruct((G, D), jnp.float32),
    scratch_shapes=[
        pltpu.VMEM((G, D), jnp.float32),
        pltpu.VMEM((G, 1), jnp.float32),
    ],
    compiler_params=pltpu.CompilerParams(
        dimension_semantics=("arbitrary",),
    ),
)


def kernel(x, edge_attr, W_cat, b_cat, W_init, b_init, W1, b1, W2, b2,
           W_out, b_out, edge_index, batch):
    del edge_attr
    row = edge_index[0]
    col = edge_index[1]

    degp = _deg_call(col)
    degs = degp[:, :, 0]

    xs1, dis = _ab_call(x, degs, W_cat.T, b_cat[None, :], W_init.T,
                        b_init[None, :], W1.T)
    acc1 = _conv_call(xs1, row, col)
    xs2 = _c_call(acc1, xs1, dis, b1[None, :], W2.T)
    acc2 = _conv_call(xs2, row, col)
    out = _d_call(acc2, xs2, dis, b2[None, :], batch.reshape(NG, 1, R),
                  W_out.T, b_out[None, :])
    return out


# speedup vs baseline: 11.2299x; 11.2299x over previous
import jax
import jax.numpy as jnp
from jax import lax
from jax.experimental import pallas as pl
from jax.experimental.pallas import tpu as pltpu
from jax.experimental.pallas import tpu_sc as plsc

N = 10000
E = 320000
D = 128
D_CAT = 32
D_CONT = 96
G = 256

NC = 2
NS = 16
NW = NC * NS
EPW = E // NW
K = 80
CH = EPW // K
RPT = 632
N_PAD = NS * RPT
DW = 16
ZR = 158

R = 2000
NG = N // R

_HI = jax.lax.Precision.HIGHEST
_mesh = plsc.VectorSubcoreMesh(core_axis_name="c", subcore_axis_name="s")
_smesh = plsc.ScalarSubcoreMesh(axis_name="c", num_cores=2)


def _deg_tec(col_hbm, out_hbm, acc):
    cid = lax.axis_index("c")
    sid = lax.axis_index("s")
    wid = cid * NS + sid

    def _scoped(colb, onesb, zb):
        def _ofill(i, _):
            onesb[i, :] = jnp.full((DW,), 1.0, jnp.float32)
            return 0

        lax.fori_loop(0, K, _ofill, 0)

        def _zfill(i, _):
            zb[i, :] = jnp.zeros((DW,), jnp.float32)
            return 0

        lax.fori_loop(0, RPT, _zfill, 0)
        pltpu.sync_copy(zb, acc.at[pl.ds(sid * RPT, RPT)])
        plsc.subcore_barrier()

        base = wid * EPW

        def _step(j, _):
            off = pl.multiple_of(base + j * K, 8)
            pltpu.sync_copy(col_hbm.at[pl.ds(off, K)], colb)
            pltpu.sync_copy(onesb, acc.at[colb], add=True)
            return 0

        lax.fori_loop(0, CH, _step, 0)
        plsc.subcore_barrier()
        pltpu.sync_copy(acc.at[pl.ds(sid * RPT, RPT)], zb)
        pltpu.sync_copy(zb, out_hbm.at[wid])

    pl.run_scoped(
        _scoped,
        pltpu.VMEM((K,), jnp.int32),
        pltpu.VMEM((K, DW), jnp.float32),
        pltpu.VMEM((RPT, DW), jnp.float32),
    )


def _deg_scs(col_hbm, out_hbm, acc):
    pass


_deg_call = pl.kernel(
    [_deg_tec, _deg_scs],
    out_type=jax.ShapeDtypeStruct((NW, RPT, DW), jnp.float32),
    mesh=[_mesh, _smesh],
    scratch_types=[
        pltpu.VMEM_SHARED((N_PAD, DW), jnp.float32),
    ],
)


def _conv_tec(xs_hbm, row_hbm, col_hbm, zeros_hbm, out_hbm, acc):
    cid = lax.axis_index("c")
    sid = lax.axis_index("s")
    wid = cid * NS + sid

    def _scoped(rowb, colb, gbuf, zb):
        pltpu.sync_copy(zeros_hbm, zb)
        for p in range(RPT // ZR):
            pltpu.sync_copy(zb, acc.at[pl.ds(sid * RPT + p * ZR, ZR)])
        plsc.subcore_barrier()

        base = wid * EPW

        def _step(j, _):
            off = pl.multiple_of(base + j * K, 8)
            pltpu.sync_copy(row_hbm.at[pl.ds(off, K)], rowb)
            pltpu.sync_copy(xs_hbm.at[rowb], gbuf)
            pltpu.sync_copy(col_hbm.at[pl.ds(off, K)], colb)
            pltpu.sync_copy(gbuf, acc.at[colb], add=True)
            return 0

        lax.fori_loop(0, CH, _step, 0)
        plsc.subcore_barrier()
        for p in range(RPT // ZR):
            pltpu.sync_copy(acc.at[pl.ds(sid * RPT + p * ZR, ZR)], zb)
            pltpu.sync_copy(zb, out_hbm.at[wid * (RPT // ZR) + p])

    pl.run_scoped(
        _scoped,
        pltpu.VMEM((K,), jnp.int32),
        pltpu.VMEM((K,), jnp.int32),
        pltpu.VMEM((K, D), jnp.float32),
        pltpu.VMEM((ZR, D), jnp.float32),
    )


def _conv_scs(xs_hbm, row_hbm, col_hbm, zeros_hbm, out_hbm, acc):
    pass


_conv_call = pl.kernel(
    [_conv_tec, _conv_scs],
    out_type=jax.ShapeDtypeStruct((NW * (RPT // ZR), ZR, D), jnp.float32),
    mesh=[_mesh, _smesh],
    scratch_types=[
        pltpu.VMEM_SHARED((N_PAD, D), jnp.float32),
    ],
)


def _elu(v):
    return jnp.where(v > 0, v, jnp.exp(jnp.minimum(v, 0.0)) - 1.0)


def _ab_body(x_ref, degs_ref, wcat_ref, bcat_ref, wi_ref, bi_ref, w1_ref,
             xs1_ref, dis_ref):
    xb = x_ref[...]
    degs = degs_ref[...]
    deg = (degs[0, 0, 0, :] + degs[1, 0, 0, :] + 1.0)[:, None]
    dis = lax.rsqrt(deg)
    s = jnp.dot(xb[:, :D_CAT], wcat_ref[...], preferred_element_type=jnp.float32,
                precision=_HI) + bcat_ref[...]
    s = jnp.maximum(s, 0.0)
    h = (jnp.dot(xb[:, D_CAT:], wi_ref[...][:D_CONT],
                 preferred_element_type=jnp.float32, precision=_HI)
         + jnp.dot(s, wi_ref[...][D_CONT:],
                   preferred_element_type=jnp.float32, precision=_HI)
         + bi_ref[...])
    h = jnp.maximum(h, 0.0)
    xs1_ref[...] = jnp.dot(h, w1_ref[...], preferred_element_type=jnp.float32,
                           precision=_HI) * dis
    dis_ref[...] = dis


_ab_call = pl.pallas_call(
    _ab_body,
    grid=(NG,),
    in_specs=[
        pl.BlockSpec((R, D), lambda i: (i, 0)),
        pl.BlockSpec((NC, 1, 1, R), lambda i: (0, i, 0, 0)),
        pl.BlockSpec((D_CAT, D_CAT), lambda i: (0, 0)),
        pl.BlockSpec((1, D_CAT), lambda i: (0, 0)),
        pl.BlockSpec((D, D), lambda i: (0, 0)),
        pl.BlockSpec((1, D), lambda i: (0, 0)),
        pl.BlockSpec((D, D), lambda i: (0, 0)),
    ],
    out_specs=[
        pl.BlockSpec((R, D), lambda i: (i, 0)),
        pl.BlockSpec((R, 1), lambda i: (i, 0)),
    ],
    out_shape=[
        jax.ShapeDtypeStruct((N, D), jnp.float32),
        jax.ShapeDtypeStruct((N, 1), jnp.float32),
    ],
)


def _c_body(acc_ref, xs_ref, dis_ref, b_ref, w_ref, out_ref):
    a = acc_ref[0] + acc_ref[1] + xs_ref[...]
    dis = dis_ref[...]
    h = _elu(a * dis + b_ref[...])
    out_ref[...] = jnp.dot(h, w_ref[...], preferred_element_type=jnp.float32,
                           precision=_HI) * dis


_c_call = pl.pallas_call(
    _c_body,
    grid=(NG,),
    in_specs=[
        pl.BlockSpec((NC, R, D), lambda i: (0, i, 0)),
        pl.BlockSpec((R, D), lambda i: (i, 0)),
        pl.BlockSpec((R, 1), lambda i: (i, 0)),
        pl.BlockSpec((1, D), lambda i: (0, 0)),
        pl.BlockSpec((D, D), lambda i: (0, 0)),
    ],
    out_specs=pl.BlockSpec((R, D), lambda i: (i, 0)),
    out_shape=jax.ShapeDtypeStruct((N, D), jnp.float32),
)


def _d_body(acc_ref, xs_ref, dis_ref, b_ref, batch_ref, wout_ref, bout_ref,
            out_ref, pooled_acc, cnt_acc):
    i = pl.program_id(0)

    @pl.when(i == 0)
    def _():
        pooled_acc[...] = jnp.zeros_like(pooled_acc)
        cnt_acc[...] = jnp.zeros_like(cnt_acc)

    a = acc_ref[0] + acc_ref[1] + xs_ref[...]
    dis = dis_ref[...]
    h = _elu(a * dis + b_ref[...])
    bb = batch_ref[0, 0, :]
    oh = (bb[:, None] == lax.broadcasted_iota(jnp.int32, (R, G), 1)
          ).astype(jnp.float32)
    pooled_acc[...] += lax.dot_general(
        oh, h, (((0,), (0,)), ((), ())), preferred_element_type=jnp.float32,
        precision=_HI)
    cnt_acc[...] += jnp.sum(oh, axis=0)[:, None]

    @pl.when(i == pl.num_programs(0) - 1)
    def _():
        pooled = pooled_acc[...] / jnp.maximum(cnt_acc[...], 1.0)
        out_ref[...] = jnp.dot(pooled, wout_ref[...],
                               preferred_element_type=jnp.float32,
                               precision=_HI) + bout_ref[...]


_d_call = pl.pallas_call(
    _d_body,
    grid=(NG,),
    in_specs=[
        pl.BlockSpec((NC, R, D), lambda i: (0, i, 0)),
        pl.BlockSpec((R, D), lambda i: (i, 0)),
        pl.BlockSpec((R, 1), lambda i: (i, 0)),
        pl.BlockSpec((1, D), lambda i: (0, 0)),
        pl.BlockSpec((1, 1, R), lambda i: (i, 0, 0)),
        pl.BlockSpec((D, D), lambda i: (0, 0)),
        pl.BlockSpec((1, D), lambda i: (0, 0)),
    ],
    out_specs=pl.BlockSpec((G, D), lambda i: (0, 0)),
    out_shape=jax.ShapeDtypeStruct((G, D), jnp.float32),
    scratch_shapes=[
        pltpu.VMEM((G, D), jnp.float32),
        pltpu.VMEM((G, 1), jnp.float32),
    ],
    compiler_params=pltpu.CompilerParams(
        dimension_semantics=("arbitrary",),
    ),
)


def kernel(x, edge_attr, W_cat, b_cat, W_init, b_init, W1, b1, W2, b2,
           W_out, b_out, edge_index, batch):
    del edge_attr
    row = edge_index[0]
    col = edge_index[1]

    degp = _deg_call(col)
    degs = (degp.reshape(NC, N_PAD, DW)[:, :N, 0]).reshape(NC, NG, 1, R)

    zrow = jnp.zeros((ZR, D), jnp.float32)

    xs1, dis = _ab_call(x, degs, W_cat.T, b_cat[None, :], W_init.T,
                        b_init[None, :], W1.T)
    acc1 = _conv_call(xs1, row, col, zrow).reshape(NC, N_PAD, D)
    xs2 = _c_call(acc1, xs1, dis, b1[None, :], W2.T)
    acc2 = _conv_call(xs2, row, col, zrow).reshape(NC, N_PAD, D)
    out = _d_call(acc2, xs2, dis, b2[None, :], batch.reshape(NG, 1, R),
                  W_out.T, b_out[None, :])
    return out


# speedup vs baseline: 20.0553x; 1.7859x over previous
import jax
import jax.numpy as jnp
from jax import lax
from jax.experimental import pallas as pl
from jax.experimental.pallas import tpu as pltpu
from jax.experimental.pallas import tpu_sc as plsc

N = 10000
E = 320000
D = 128
D_CAT = 32
D_CONT = 96
G = 256

NC = 2
NS = 16
NW = NC * NS
EPW = E // NW
K = 80
CH = EPW // K
CH2 = 25
NB = CH // CH2
CHT = E // K
RPT = 632
N_PAD = NS * RPT
DW = 16
ZR = 158

R = 2000
NG = N // R

_HI = jax.lax.Precision.HIGHEST
_mesh = plsc.VectorSubcoreMesh(core_axis_name="c", subcore_axis_name="s")
_smesh = plsc.ScalarSubcoreMesh(axis_name="c", num_cores=2)


def _deg_tec(col_hbm, out_hbm, acc):
    cid = lax.axis_index("c")
    sid = lax.axis_index("s")
    wid = cid * NS + sid

    def _scoped(colb, onesb, zb):
        def _ofill(i, _):
            onesb[i, :] = jnp.full((DW,), 1.0, jnp.float32)
            return 0

        lax.fori_loop(0, K, _ofill, 0)

        def _zfill(i, _):
            zb[i, :] = jnp.zeros((DW,), jnp.float32)
            return 0

        lax.fori_loop(0, RPT, _zfill, 0)
        pltpu.sync_copy(zb, acc.at[pl.ds(sid * RPT, RPT)])
        plsc.subcore_barrier()

        base = wid * EPW

        def _step(j, _):
            off = pl.multiple_of(base + j * K, 8)
            pltpu.sync_copy(col_hbm.at[pl.ds(off, K)], colb)
            pltpu.sync_copy(onesb, acc.at[colb], add=True)
            return 0

        lax.fori_loop(0, CH, _step, 0)
        plsc.subcore_barrier()
        pltpu.sync_copy(acc.at[pl.ds(sid * RPT, RPT)], zb)
        pltpu.sync_copy(zb, out_hbm.at[wid])

    pl.run_scoped(
        _scoped,
        pltpu.VMEM((K,), jnp.int32),
        pltpu.VMEM((K, DW), jnp.float32),
        pltpu.VMEM((RPT, DW), jnp.float32),
    )


def _deg_scs(col_hbm, out_hbm, acc):
    pass


_deg_call = pl.kernel(
    [_deg_tec, _deg_scs],
    out_type=jax.ShapeDtypeStruct((NW, RPT, DW), jnp.float32),
    mesh=[_mesh, _smesh],
    scratch_types=[
        pltpu.VMEM_SHARED((N_PAD, DW), jnp.float32),
    ],
)


def _conv_tec(xs_hbm, row_hbm, col_hbm, zeros_hbm, out_hbm, acc):
    cid = lax.axis_index("c")
    sid = lax.axis_index("s")
    wid = cid * NS + sid

    def _scoped(rowb, colb, gbuf0, gbuf1, zb, gsem, ssem):
        pltpu.sync_copy(zeros_hbm, zb)
        for p in range(RPT // ZR):
            pltpu.sync_copy(zb, acc.at[pl.ds(sid * RPT + p * ZR, ZR)])
        plsc.subcore_barrier()

        gbufs = (gbuf0, gbuf1)
        for b in range(NB):
            blk = wid * NB + b
            pltpu.sync_copy(row_hbm.at[blk], rowb)
            pltpu.sync_copy(col_hbm.at[blk], colb)
            pend_g = None
            pend_s = [None, None]
            pend_j = -1
            for j in range(CH2):
                buf = gbufs[j % 2]
                if pend_s[j % 2] is not None:
                    pend_s[j % 2].wait()
                    pend_s[j % 2] = None
                g = pltpu.async_copy(xs_hbm.at[rowb.at[j]], buf, gsem)
                if pend_g is not None:
                    pend_g.wait()
                    pend_s[pend_j % 2] = pltpu.async_copy(
                        gbufs[pend_j % 2], acc.at[colb.at[pend_j]], ssem,
                        add=True)
                pend_g = g
                pend_j = j
            pend_g.wait()
            last_s = pltpu.async_copy(
                gbufs[pend_j % 2], acc.at[colb.at[pend_j]], ssem, add=True)
            for s in pend_s:
                if s is not None:
                    s.wait()
            last_s.wait()
        plsc.subcore_barrier()
        for p in range(RPT // ZR):
            pltpu.sync_copy(acc.at[pl.ds(sid * RPT + p * ZR, ZR)], zb)
            pltpu.sync_copy(zb, out_hbm.at[wid * (RPT // ZR) + p])

    pl.run_scoped(
        _scoped,
        pltpu.VMEM((CH2, K), jnp.int32),
        pltpu.VMEM((CH2, K), jnp.int32),
        pltpu.VMEM((K, D), jnp.float32),
        pltpu.VMEM((K, D), jnp.float32),
        pltpu.VMEM((ZR, D), jnp.float32),
        pltpu.SemaphoreType.DMA(()),
        pltpu.SemaphoreType.DMA(()),
    )


def _conv_scs(xs_hbm, row_hbm, col_hbm, zeros_hbm, out_hbm, acc):
    pass


_conv_call = pl.kernel(
    [_conv_tec, _conv_scs],
    out_type=jax.ShapeDtypeStruct((NW * (RPT // ZR), ZR, D), jnp.float32),
    mesh=[_mesh, _smesh],
    scratch_types=[
        pltpu.VMEM_SHARED((N_PAD, D), jnp.float32),
    ],
)


def _elu(v):
    return jnp.where(v > 0, v, jnp.exp(jnp.minimum(v, 0.0)) - 1.0)


def _ab_body(x_ref, degs_ref, wcat_ref, bcat_ref, wi_ref, bi_ref, w1_ref,
             xs1_ref, dis_ref):
    xb = x_ref[...]
    degs = degs_ref[...]
    deg = (degs[0, 0, 0, :] + degs[1, 0, 0, :] + 1.0)[:, None]
    dis = lax.rsqrt(deg)
    s = jnp.dot(xb[:, :D_CAT], wcat_ref[...], preferred_element_type=jnp.float32,
                precision=_HI) + bcat_ref[...]
    s = jnp.maximum(s, 0.0)
    h = (jnp.dot(xb[:, D_CAT:], wi_ref[...][:D_CONT],
                 preferred_element_type=jnp.float32, precision=_HI)
         + jnp.dot(s, wi_ref[...][D_CONT:],
                   preferred_element_type=jnp.float32, precision=_HI)
         + bi_ref[...])
    h = jnp.maximum(h, 0.0)
    xs1_ref[...] = jnp.dot(h, w1_ref[...], preferred_element_type=jnp.float32,
                           precision=_HI) * dis
    dis_ref[...] = dis


_ab_call = pl.pallas_call(
    _ab_body,
    grid=(NG,),
    in_specs=[
        pl.BlockSpec((R, D), lambda i: (i, 0)),
        pl.BlockSpec((NC, 1, 1, R), lambda i: (0, i, 0, 0)),
        pl.BlockSpec((D_CAT, D_CAT), lambda i: (0, 0)),
        pl.BlockSpec((1, D_CAT), lambda i: (0, 0)),
        pl.BlockSpec((D, D), lambda i: (0, 0)),
        pl.BlockSpec((1, D), lambda i: (0, 0)),
        pl.BlockSpec((D, D), lambda i: (0, 0)),
    ],
    out_specs=[
        pl.BlockSpec((R, D), lambda i: (i, 0)),
        pl.BlockSpec((R, 1), lambda i: (i, 0)),
    ],
    out_shape=[
        jax.ShapeDtypeStruct((N, D), jnp.float32),
        jax.ShapeDtypeStruct((N, 1), jnp.float32),
    ],
)


def _c_body(acc_ref, xs_ref, dis_ref, b_ref, w_ref, out_ref):
    a = acc_ref[0] + acc_ref[1] + xs_ref[...]
    dis = dis_ref[...]
    h = _elu(a * dis + b_ref[...])
    out_ref[...] = jnp.dot(h, w_ref[...], preferred_element_type=jnp.float32,
                           precision=_HI) * dis


_c_call = pl.pallas_call(
    _c_body,
    grid=(NG,),
    in_specs=[
        pl.BlockSpec((NC, R, D), lambda i: (0, i, 0)),
        pl.BlockSpec((R, D), lambda i: (i, 0)),
        pl.BlockSpec((R, 1), lambda i: (i, 0)),
        pl.BlockSpec((1, D), lambda i: (0, 0)),
        pl.BlockSpec((D, D), lambda i: (0, 0)),
    ],
    out_specs=pl.BlockSpec((R, D), lambda i: (i, 0)),
    out_shape=jax.ShapeDtypeStruct((N, D), jnp.float32),
)


def _d_body(acc_ref, xs_ref, dis_ref, b_ref, batch_ref, wout_ref, bout_ref,
            out_ref, pooled_acc, cnt_acc):
    i = pl.program_id(0)

    @pl.when(i == 0)
    def _():
        pooled_acc[...] = jnp.zeros_like(pooled_acc)
        cnt_acc[...] = jnp.zeros_like(cnt_acc)

    a = acc_ref[0] + acc_ref[1] + xs_ref[...]
    dis = dis_ref[...]
    h = _elu(a * dis + b_ref[...])
    bb = batch_ref[0, 0, :]
    oh = (bb[:, None] == lax.broadcasted_iota(jnp.int32, (R, G), 1)
          ).astype(jnp.float32)
    pooled_acc[...] += lax.dot_general(
        oh, h, (((0,), (0,)), ((), ())), preferred_element_type=jnp.float32,
        precision=_HI)
    cnt_acc[...] += jnp.sum(oh, axis=0)[:, None]

    @pl.when(i == pl.num_programs(0) - 1)
    def _():
        pooled = pooled_acc[...] / jnp.maximum(cnt_acc[...], 1.0)
        out_ref[...] = jnp.dot(pooled, wout_ref[...],
                               preferred_element_type=jnp.float32,
                               precision=_HI) + bout_ref[...]


_d_call = pl.pallas_call(
    _d_body,
    grid=(NG,),
    in_specs=[
        pl.BlockSpec((NC, R, D), lambda i: (0, i, 0)),
        pl.BlockSpec((R, D), lambda i: (i, 0)),
        pl.BlockSpec((R, 1), lambda i: (i, 0)),
        pl.BlockSpec((1, D), lambda i: (0, 0)),
        pl.BlockSpec((1, 1, R), lambda i: (i, 0, 0)),
        pl.BlockSpec((D, D), lambda i: (0, 0)),
        pl.BlockSpec((1, D), lambda i: (0, 0)),
    ],
    out_specs=pl.BlockSpec((G, D), lambda i: (0, 0)),
    out_shape=jax.ShapeDtypeStruct((G, D), jnp.float32),
    scratch_shapes=[
        pltpu.VMEM((G, D), jnp.float32),
        pltpu.VMEM((G, 1), jnp.float32),
    ],
    compiler_params=pltpu.CompilerParams(
        dimension_semantics=("arbitrary",),
    ),
)


def kernel(x, edge_attr, W_cat, b_cat, W_init, b_init, W1, b1, W2, b2,
           W_out, b_out, edge_index, batch):
    del edge_attr
    row = edge_index[0]
    col = edge_index[1]

    degp = _deg_call(col)
    degs = (degp.reshape(NC, N_PAD, DW)[:, :N, 0]).reshape(NC, NG, 1, R)

    zrow = jnp.zeros((ZR, D), jnp.float32)
    row2 = row.reshape(CHT // CH2, CH2, K)
    col2 = col.reshape(CHT // CH2, CH2, K)

    xs1, dis = _ab_call(x, degs, W_cat.T, b_cat[None, :], W_init.T,
                        b_init[None, :], W1.T)
    acc1 = _conv_call(xs1, row2, col2, zrow).reshape(NC, N_PAD, D)
    xs2 = _c_call(acc1, xs1, dis, b1[None, :], W2.T)
    acc2 = _conv_call(xs2, row2, col2, zrow).reshape(NC, N_PAD, D)
    out = _d_call(acc2, xs2, dis, b2[None, :], batch.reshape(NG, 1, R),
                  W_out.T, b_out[None, :])
    return out


# speedup vs baseline: 22.7532x; 1.1345x over previous
import jax
import jax.numpy as jnp
from jax import lax
from jax.experimental import pallas as pl
from jax.experimental.pallas import tpu as pltpu
from jax.experimental.pallas import tpu_sc as plsc

N = 10000
E = 320000
D = 128
D_CAT = 32
D_CONT = 96
G = 256

NC = 2
NS = 16
NW = NC * NS
EPW = E // NW
K = 80
CH = EPW // K
CH2 = 25
NB = CH // CH2
CHT = E // K
RPT = 632
N_PAD = NS * RPT
DW = 16
ZR = 158

R = 2000
NG = N // R

_HI = jax.lax.Precision.HIGHEST
_mesh = plsc.VectorSubcoreMesh(core_axis_name="c", subcore_axis_name="s")
_smesh = plsc.ScalarSubcoreMesh(axis_name="c", num_cores=2)


def _deg_tec(col_hbm, out_hbm, acc):
    cid = lax.axis_index("c")
    sid = lax.axis_index("s")
    wid = cid * NS + sid

    def _scoped(colb, onesb, zb, ssem):
        def _ofill(i, _):
            onesb[i, :] = jnp.full((DW,), 1.0, jnp.float32)
            return 0

        lax.fori_loop(0, K, _ofill, 0)

        def _zfill(i, _):
            zb[i, :] = jnp.zeros((DW,), jnp.float32)
            return 0

        lax.fori_loop(0, RPT, _zfill, 0)
        pltpu.sync_copy(zb, acc.at[pl.ds(sid * RPT, RPT)])
        plsc.subcore_barrier()

        for b in range(NB):
            pltpu.sync_copy(col_hbm.at[wid * NB + b], colb)
            pend = [
                pltpu.async_copy(onesb, acc.at[colb.at[j]], ssem, add=True)
                for j in range(CH2)
            ]
            for s in pend:
                s.wait()
        plsc.subcore_barrier()
        pltpu.sync_copy(acc.at[pl.ds(sid * RPT, RPT)], zb)
        pltpu.sync_copy(zb, out_hbm.at[wid])

    pl.run_scoped(
        _scoped,
        pltpu.VMEM((CH2, K), jnp.int32),
        pltpu.VMEM((K, DW), jnp.float32),
        pltpu.VMEM((RPT, DW), jnp.float32),
        pltpu.SemaphoreType.DMA(()),
    )


def _deg_scs(col_hbm, out_hbm, acc):
    pass


_deg_call = pl.kernel(
    [_deg_tec, _deg_scs],
    out_type=jax.ShapeDtypeStruct((NW, RPT, DW), jnp.float32),
    mesh=[_mesh, _smesh],
    scratch_types=[
        pltpu.VMEM_SHARED((N_PAD, DW), jnp.float32),
    ],
)


def _conv_tec(xs_hbm, row_hbm, col_hbm, zeros_hbm, out_hbm, acc):
    cid = lax.axis_index("c")
    sid = lax.axis_index("s")
    wid = cid * NS + sid

    def _scoped(rowb, colb, gbuf0, gbuf1, zb, gsem, ssem):
        pltpu.sync_copy(zeros_hbm, zb)
        for p in range(RPT // ZR):
            pltpu.sync_copy(zb, acc.at[pl.ds(sid * RPT + p * ZR, ZR)])
        plsc.subcore_barrier()

        gbufs = (gbuf0, gbuf1)
        for b in range(NB):
            blk = wid * NB + b
            pltpu.sync_copy(row_hbm.at[blk], rowb)
            pltpu.sync_copy(col_hbm.at[blk], colb)
            pend_g = None
            pend_s = [None, None]
            pend_j = -1
            for j in range(CH2):
                buf = gbufs[j % 2]
                if pend_s[j % 2] is not None:
                    pend_s[j % 2].wait()
                    pend_s[j % 2] = None
                g = pltpu.async_copy(xs_hbm.at[rowb.at[j]], buf, gsem)
                if pend_g is not None:
                    pend_g.wait()
                    pend_s[pend_j % 2] = pltpu.async_copy(
                        gbufs[pend_j % 2], acc.at[colb.at[pend_j]], ssem,
                        add=True)
                pend_g = g
                pend_j = j
            pend_g.wait()
            last_s = pltpu.async_copy(
                gbufs[pend_j % 2], acc.at[colb.at[pend_j]], ssem, add=True)
            for s in pend_s:
                if s is not None:
                    s.wait()
            last_s.wait()
        plsc.subcore_barrier()
        for p in range(RPT // ZR):
            pltpu.sync_copy(acc.at[pl.ds(sid * RPT + p * ZR, ZR)], zb)
            pltpu.sync_copy(zb, out_hbm.at[wid * (RPT // ZR) + p])

    pl.run_scoped(
        _scoped,
        pltpu.VMEM((CH2, K), jnp.int32),
        pltpu.VMEM((CH2, K), jnp.int32),
        pltpu.VMEM((K, D), jnp.float32),
        pltpu.VMEM((K, D), jnp.float32),
        pltpu.VMEM((ZR, D), jnp.float32),
        pltpu.SemaphoreType.DMA(()),
        pltpu.SemaphoreType.DMA(()),
    )


def _conv_scs(xs_hbm, row_hbm, col_hbm, zeros_hbm, out_hbm, acc):
    pass


_conv_call = pl.kernel(
    [_conv_tec, _conv_scs],
    out_type=jax.ShapeDtypeStruct((NW * (RPT // ZR), ZR, D), jnp.float32),
    mesh=[_mesh, _smesh],
    scratch_types=[
        pltpu.VMEM_SHARED((N_PAD, D), jnp.float32),
    ],
)


def _elu(v):
    return jnp.where(v > 0, v, jnp.exp(jnp.minimum(v, 0.0)) - 1.0)


def _ab_body(x_ref, degs_ref, wcat_ref, bcat_ref, wi_ref, bi_ref, w1_ref,
             xs1_ref, dis_ref):
    xb = x_ref[...]
    degs = degs_ref[...]
    deg = (degs[0, 0, 0, :] + degs[1, 0, 0, :] + 1.0)[:, None]
    dis = lax.rsqrt(deg)
    s = jnp.dot(xb[:, :D_CAT], wcat_ref[...], preferred_element_type=jnp.float32,
                precision=_HI) + bcat_ref[...]
    s = jnp.maximum(s, 0.0)
    h = (jnp.dot(xb[:, D_CAT:], wi_ref[...][:D_CONT],
                 preferred_element_type=jnp.float32, precision=_HI)
         + jnp.dot(s, wi_ref[...][D_CONT:],
                   preferred_element_type=jnp.float32, precision=_HI)
         + bi_ref[...])
    h = jnp.maximum(h, 0.0)
    xs1_ref[...] = jnp.dot(h, w1_ref[...], preferred_element_type=jnp.float32,
                           precision=_HI) * dis
    dis_ref[...] = dis


_ab_call = pl.pallas_call(
    _ab_body,
    grid=(NG,),
    in_specs=[
        pl.BlockSpec((R, D), lambda i: (i, 0)),
        pl.BlockSpec((NC, 1, 1, R), lambda i: (0, i, 0, 0)),
        pl.BlockSpec((D_CAT, D_CAT), lambda i: (0, 0)),
        pl.BlockSpec((1, D_CAT), lambda i: (0, 0)),
        pl.BlockSpec((D, D), lambda i: (0, 0)),
        pl.BlockSpec((1, D), lambda i: (0, 0)),
        pl.BlockSpec((D, D), lambda i: (0, 0)),
    ],
    out_specs=[
        pl.BlockSpec((R, D), lambda i: (i, 0)),
        pl.BlockSpec((R, 1), lambda i: (i, 0)),
    ],
    out_shape=[
        jax.ShapeDtypeStruct((N, D), jnp.float32),
        jax.ShapeDtypeStruct((N, 1), jnp.float32),
    ],
)


def _c_body(acc_ref, xs_ref, dis_ref, b_ref, w_ref, out_ref):
    a = acc_ref[0] + acc_ref[1] + xs_ref[...]
    dis = dis_ref[...]
    h = _elu(a * dis + b_ref[...])
    out_ref[...] = jnp.dot(h, w_ref[...], preferred_element_type=jnp.float32,
                           precision=_HI) * dis


_c_call = pl.pallas_call(
    _c_body,
    grid=(NG,),
    in_specs=[
        pl.BlockSpec((NC, R, D), lambda i: (0, i, 0)),
        pl.BlockSpec((R, D), lambda i: (i, 0)),
        pl.BlockSpec((R, 1), lambda i: (i, 0)),
        pl.BlockSpec((1, D), lambda i: (0, 0)),
        pl.BlockSpec((D, D), lambda i: (0, 0)),
    ],
    out_specs=pl.BlockSpec((R, D), lambda i: (i, 0)),
    out_shape=jax.ShapeDtypeStruct((N, D), jnp.float32),
)


def _d_body(acc_ref, xs_ref, dis_ref, b_ref, batch_ref, wout_ref, bout_ref,
            out_ref, pooled_acc, cnt_acc):
    i = pl.program_id(0)

    @pl.when(i == 0)
    def _():
        pooled_acc[...] = jnp.zeros_like(pooled_acc)
        cnt_acc[...] = jnp.zeros_like(cnt_acc)

    a = acc_ref[0] + acc_ref[1] + xs_ref[...]
    dis = dis_ref[...]
    h = _elu(a * dis + b_ref[...])
    bb = batch_ref[0, 0, :]
    oh = (bb[:, None] == lax.broadcasted_iota(jnp.int32, (R, G), 1)
          ).astype(jnp.float32)
    pooled_acc[...] += lax.dot_general(
        oh, h, (((0,), (0,)), ((), ())), preferred_element_type=jnp.float32,
        precision=_HI)
    cnt_acc[...] += jnp.sum(oh, axis=0)[:, None]

    @pl.when(i == pl.num_programs(0) - 1)
    def _():
        pooled = pooled_acc[...] / jnp.maximum(cnt_acc[...], 1.0)
        out_ref[...] = jnp.dot(pooled, wout_ref[...],
                               preferred_element_type=jnp.float32,
                               precision=_HI) + bout_ref[...]


_d_call = pl.pallas_call(
    _d_body,
    grid=(NG,),
    in_specs=[
        pl.BlockSpec((NC, R, D), lambda i: (0, i, 0)),
        pl.BlockSpec((R, D), lambda i: (i, 0)),
        pl.BlockSpec((R, 1), lambda i: (i, 0)),
        pl.BlockSpec((1, D), lambda i: (0, 0)),
        pl.BlockSpec((1, 1, R), lambda i: (i, 0, 0)),
        pl.BlockSpec((D, D), lambda i: (0, 0)),
        pl.BlockSpec((1, D), lambda i: (0, 0)),
    ],
    out_specs=pl.BlockSpec((G, D), lambda i: (0, 0)),
    out_shape=jax.ShapeDtypeStruct((G, D), jnp.float32),
    scratch_shapes=[
        pltpu.VMEM((G, D), jnp.float32),
        pltpu.VMEM((G, 1), jnp.float32),
    ],
    compiler_params=pltpu.CompilerParams(
        dimension_semantics=("arbitrary",),
    ),
)


def kernel(x, edge_attr, W_cat, b_cat, W_init, b_init, W1, b1, W2, b2,
           W_out, b_out, edge_index, batch):
    del edge_attr
    row = edge_index[0]
    col = edge_index[1]

    col2 = col.reshape(CHT // CH2, CH2, K)
    degp = _deg_call(col2)
    degs = (degp.reshape(NC, N_PAD, DW)[:, :N, 0]).reshape(NC, NG, 1, R)

    zrow = jnp.zeros((ZR, D), jnp.float32)
    row2 = row.reshape(CHT // CH2, CH2, K)

    xs1, dis = _ab_call(x, degs, W_cat.T, b_cat[None, :], W_init.T,
                        b_init[None, :], W1.T)
    acc1 = _conv_call(xs1, row2, col2, zrow).reshape(NC, N_PAD, D)
    xs2 = _c_call(acc1, xs1, dis, b1[None, :], W2.T)
    acc2 = _conv_call(xs2, row2, col2, zrow).reshape(NC, N_PAD, D)
    out = _d_call(acc2, xs2, dis, b2[None, :], batch.reshape(NG, 1, R),
                  W_out.T, b_out[None, :])
    return out


# speedup vs baseline: 24.7830x; 1.0892x over previous
import jax
import jax.numpy as jnp
from jax import lax
from jax.experimental import pallas as pl
from jax.experimental.pallas import tpu as pltpu
from jax.experimental.pallas import tpu_sc as plsc

N = 10000
E = 320000
D = 128
D_CAT = 32
D_CONT = 96
G = 256

NC = 2
NS = 16
NW = NC * NS
EPW = E // NW
K = 80
CH = EPW // K
CH2 = 25
NB = CH // CH2
CHT = E // K
RPT = 632
N_PAD = NS * RPT
DW = 16
ZR = 158

R = 2000
NG = N // R

_HI = jax.lax.Precision.HIGHEST
_mesh = plsc.VectorSubcoreMesh(core_axis_name="c", subcore_axis_name="s")
_smesh = plsc.ScalarSubcoreMesh(axis_name="c", num_cores=2)


def _deg_tec(col_hbm, out_hbm, acc):
    cid = lax.axis_index("c")
    sid = lax.axis_index("s")
    wid = cid * NS + sid

    def _scoped(colb, onesb, zb, ssem):
        def _ofill(i, _):
            onesb[i, :] = jnp.full((DW,), 1.0, jnp.float32)
            return 0

        lax.fori_loop(0, K, _ofill, 0)

        def _zfill(i, _):
            zb[i, :] = jnp.zeros((DW,), jnp.float32)
            return 0

        lax.fori_loop(0, RPT, _zfill, 0)
        pltpu.sync_copy(zb, acc.at[pl.ds(sid * RPT, RPT)])
        plsc.subcore_barrier()

        for b in range(NB):
            pltpu.sync_copy(col_hbm.at[wid * NB + b], colb)
            pend = [
                pltpu.async_copy(onesb, acc.at[colb.at[j]], ssem, add=True)
                for j in range(CH2)
            ]
            for s in pend:
                s.wait()
        plsc.subcore_barrier()
        pltpu.sync_copy(acc.at[pl.ds(sid * RPT, RPT)], zb)
        pltpu.sync_copy(zb, out_hbm.at[wid])

    pl.run_scoped(
        _scoped,
        pltpu.VMEM((CH2, K), jnp.int32),
        pltpu.VMEM((K, DW), jnp.float32),
        pltpu.VMEM((RPT, DW), jnp.float32),
        pltpu.SemaphoreType.DMA(()),
    )


def _deg_scs(col_hbm, out_hbm, acc):
    pass


_deg_call = pl.kernel(
    [_deg_tec, _deg_scs],
    out_type=jax.ShapeDtypeStruct((NW, RPT, DW), jnp.float32),
    mesh=[_mesh, _smesh],
    scratch_types=[
        pltpu.VMEM_SHARED((N_PAD, DW), jnp.float32),
    ],
)


def _conv_tec(xs_hbm, row_hbm, col_hbm, zeros_hbm, out_hbm, acc):
    cid = lax.axis_index("c")
    sid = lax.axis_index("s")
    wid = cid * NS + sid

    def _scoped(rowb, colb, gbuf0, gbuf1, zb, gsem, ssem):
        pltpu.sync_copy(zeros_hbm, zb)
        for p in range(RPT // ZR):
            pltpu.sync_copy(zb, acc.at[pl.ds(sid * RPT + p * ZR, ZR)])
        plsc.subcore_barrier()

        gbufs = (gbuf0, gbuf1)
        for b in range(NB):
            blk = wid * NB + b
            pltpu.sync_copy(row_hbm.at[blk], rowb)
            pltpu.sync_copy(col_hbm.at[blk], colb)
            pend_g = None
            pend_s = [None, None]
            pend_j = -1
            for j in range(CH2):
                buf = gbufs[j % 2]
                if pend_s[j % 2] is not None:
                    pend_s[j % 2].wait()
                    pend_s[j % 2] = None
                g = pltpu.async_copy(xs_hbm.at[rowb.at[j]], buf, gsem)
                if pend_g is not None:
                    pend_g.wait()
                    pend_s[pend_j % 2] = pltpu.async_copy(
                        gbufs[pend_j % 2], acc.at[colb.at[pend_j]], ssem,
                        add=True)
                pend_g = g
                pend_j = j
            pend_g.wait()
            last_s = pltpu.async_copy(
                gbufs[pend_j % 2], acc.at[colb.at[pend_j]], ssem, add=True)
            for s in pend_s:
                if s is not None:
                    s.wait()
            last_s.wait()
        plsc.subcore_barrier()
        for p in range(RPT // ZR):
            pltpu.sync_copy(acc.at[pl.ds(sid * RPT + p * ZR, ZR)], zb)
            pltpu.sync_copy(zb, out_hbm.at[wid * (RPT // ZR) + p])

    pl.run_scoped(
        _scoped,
        pltpu.VMEM((CH2, K), jnp.int32),
        pltpu.VMEM((CH2, K), jnp.int32),
        pltpu.VMEM((K, D), jnp.float32),
        pltpu.VMEM((K, D), jnp.float32),
        pltpu.VMEM((ZR, D), jnp.float32),
        pltpu.SemaphoreType.DMA(()),
        pltpu.SemaphoreType.DMA(()),
    )


def _conv_scs(xs_hbm, row_hbm, col_hbm, zeros_hbm, out_hbm, acc):
    pass


_conv_call = pl.kernel(
    [_conv_tec, _conv_scs],
    out_type=jax.ShapeDtypeStruct((NW * (RPT // ZR), ZR, D), jnp.float32),
    mesh=[_mesh, _smesh],
    scratch_types=[
        pltpu.VMEM_SHARED((N_PAD, D), jnp.float32),
    ],
)


def _elu(v):
    return jnp.where(v > 0, v, jnp.exp(jnp.minimum(v, 0.0)) - 1.0)


def _a_body(x_ref, wcat_ref, bcat_ref, wi_ref, bi_ref, h_ref):
    xb = x_ref[...]
    s = jnp.dot(xb[:, :D_CAT], wcat_ref[...], preferred_element_type=jnp.float32,
                precision=_HI) + bcat_ref[...]
    s = jnp.maximum(s, 0.0)
    h = (jnp.dot(xb[:, D_CAT:], wi_ref[...][:D_CONT],
                 preferred_element_type=jnp.float32, precision=_HI)
         + jnp.dot(s, wi_ref[...][D_CONT:],
                   preferred_element_type=jnp.float32, precision=_HI)
         + bi_ref[...])
    h_ref[...] = jnp.maximum(h, 0.0)


_a_call = pl.pallas_call(
    _a_body,
    grid=(NG,),
    in_specs=[
        pl.BlockSpec((R, D), lambda i: (i, 0)),
        pl.BlockSpec((D_CAT, D_CAT), lambda i: (0, 0)),
        pl.BlockSpec((1, D_CAT), lambda i: (0, 0)),
        pl.BlockSpec((D, D), lambda i: (0, 0)),
        pl.BlockSpec((1, D), lambda i: (0, 0)),
    ],
    out_specs=pl.BlockSpec((R, D), lambda i: (i, 0)),
    out_shape=jax.ShapeDtypeStruct((N, D), jnp.float32),
)


def _b_body(h_ref, degp_ref, w1_ref, xs1_ref, dis_ref):
    degp = degp_ref[...]
    deg = (degp[0, :, 0] + degp[1, :, 0] + 1.0)[:, None]
    dis = lax.rsqrt(deg)
    xs1_ref[...] = jnp.dot(h_ref[...], w1_ref[...],
                           preferred_element_type=jnp.float32,
                           precision=_HI) * dis
    dis_ref[...] = dis


_b_call = pl.pallas_call(
    _b_body,
    grid=(NG,),
    in_specs=[
        pl.BlockSpec((R, D), lambda i: (i, 0)),
        pl.BlockSpec((NC, R, DW), lambda i: (0, i, 0)),
        pl.BlockSpec((D, D), lambda i: (0, 0)),
    ],
    out_specs=[
        pl.BlockSpec((R, D), lambda i: (i, 0)),
        pl.BlockSpec((R, 1), lambda i: (i, 0)),
    ],
    out_shape=[
        jax.ShapeDtypeStruct((N, D), jnp.float32),
        jax.ShapeDtypeStruct((N, 1), jnp.float32),
    ],
)


def _c_body(acc_ref, xs_ref, dis_ref, b_ref, w_ref, out_ref):
    a = acc_ref[0] + acc_ref[1] + xs_ref[...]
    dis = dis_ref[...]
    h = _elu(a * dis + b_ref[...])
    out_ref[...] = jnp.dot(h, w_ref[...], preferred_element_type=jnp.float32,
                           precision=_HI) * dis


_c_call = pl.pallas_call(
    _c_body,
    grid=(NG,),
    in_specs=[
        pl.BlockSpec((NC, R, D), lambda i: (0, i, 0)),
        pl.BlockSpec((R, D), lambda i: (i, 0)),
        pl.BlockSpec((R, 1), lambda i: (i, 0)),
        pl.BlockSpec((1, D), lambda i: (0, 0)),
        pl.BlockSpec((D, D), lambda i: (0, 0)),
    ],
    out_specs=pl.BlockSpec((R, D), lambda i: (i, 0)),
    out_shape=jax.ShapeDtypeStruct((N, D), jnp.float32),
)


def _d_body(acc_ref, xs_ref, dis_ref, b_ref, batch_ref, wout_ref, bout_ref,
            out_ref, pooled_acc, cnt_acc):
    i = pl.program_id(0)

    @pl.when(i == 0)
    def _():
        pooled_acc[...] = jnp.zeros_like(pooled_acc)
        cnt_acc[...] = jnp.zeros_like(cnt_acc)

    a = acc_ref[0] + acc_ref[1] + xs_ref[...]
    dis = dis_ref[...]
    h = _elu(a * dis + b_ref[...])
    bb = batch_ref[0, 0, :]
    oh = (bb[:, None] == lax.broadcasted_iota(jnp.int32, (R, G), 1)
          ).astype(jnp.float32)
    pooled_acc[...] += lax.dot_general(
        oh, h, (((0,), (0,)), ((), ())), preferred_element_type=jnp.float32,
        precision=_HI)
    cnt_acc[...] += jnp.sum(oh, axis=0)[:, None]

    @pl.when(i == pl.num_programs(0) - 1)
    def _():
        pooled = pooled_acc[...] / jnp.maximum(cnt_acc[...], 1.0)
        out_ref[...] = jnp.dot(pooled, wout_ref[...],
                               preferred_element_type=jnp.float32,
                               precision=_HI) + bout_ref[...]


_d_call = pl.pallas_call(
    _d_body,
    grid=(NG,),
    in_specs=[
        pl.BlockSpec((NC, R, D), lambda i: (0, i, 0)),
        pl.BlockSpec((R, D), lambda i: (i, 0)),
        pl.BlockSpec((R, 1), lambda i: (i, 0)),
        pl.BlockSpec((1, D), lambda i: (0, 0)),
        pl.BlockSpec((1, 1, R), lambda i: (i, 0, 0)),
        pl.BlockSpec((D, D), lambda i: (0, 0)),
        pl.BlockSpec((1, D), lambda i: (0, 0)),
    ],
    out_specs=pl.BlockSpec((G, D), lambda i: (0, 0)),
    out_shape=jax.ShapeDtypeStruct((G, D), jnp.float32),
    scratch_shapes=[
        pltpu.VMEM((G, D), jnp.float32),
        pltpu.VMEM((G, 1), jnp.float32),
    ],
    compiler_params=pltpu.CompilerParams(
        dimension_semantics=("arbitrary",),
    ),
)


def kernel(x, edge_attr, W_cat, b_cat, W_init, b_init, W1, b1, W2, b2,
           W_out, b_out, edge_index, batch):
    del edge_attr
    row = edge_index[0]
    col = edge_index[1]

    col2 = col.reshape(CHT // CH2, CH2, K)
    degp = _deg_call(col2).reshape(NC, N_PAD, DW)

    zrow = jnp.zeros((ZR, D), jnp.float32)
    row2 = row.reshape(CHT // CH2, CH2, K)

    h = _a_call(x, W_cat.T, b_cat[None, :], W_init.T, b_init[None, :])
    xs1, dis = _b_call(h, degp, W1.T)
    acc1 = _conv_call(xs1, row2, col2, zrow).reshape(NC, N_PAD, D)
    xs2 = _c_call(acc1, xs1, dis, b1[None, :], W2.T)
    acc2 = _conv_call(xs2, row2, col2, zrow).reshape(NC, N_PAD, D)
    out = _d_call(acc2, xs2, dis, b2[None, :], batch.reshape(NG, 1, R),
                  W_out.T, b_out[None, :])
    return out


# speedup vs baseline: 24.8310x; 1.0019x over previous
import jax
import jax.numpy as jnp
from jax import lax
from jax.experimental import pallas as pl
from jax.experimental.pallas import tpu as pltpu
from jax.experimental.pallas import tpu_sc as plsc

N = 10000
E = 320000
D = 128
D_CAT = 32
D_CONT = 96
G = 256

NC = 2
NS = 16
NW = NC * NS
EPW = E // NW
K = 80
CH = EPW // K
CH2 = 25
NB = CH // CH2
CHT = E // K
RPT = 632
N_PAD = NS * RPT
DW = 16
ZR = 158

R = 2000
NG = N // R

_HI = jax.lax.Precision.HIGHEST
_mesh = plsc.VectorSubcoreMesh(core_axis_name="c", subcore_axis_name="s")
_smesh = plsc.ScalarSubcoreMesh(axis_name="c", num_cores=2)


def _deg_tec(col_hbm, out_hbm, acc):
    cid = lax.axis_index("c")
    sid = lax.axis_index("s")
    wid = cid * NS + sid

    def _scoped(colb, onesb, zb, ssem):
        def _ofill(i, _):
            onesb[i, :] = jnp.full((DW,), 1.0, jnp.float32)
            return 0

        lax.fori_loop(0, K, _ofill, 0)

        def _zfill(i, _):
            zb[i, :] = jnp.zeros((DW,), jnp.float32)
            return 0

        lax.fori_loop(0, RPT, _zfill, 0)
        pltpu.sync_copy(zb, acc.at[pl.ds(sid * RPT, RPT)])
        plsc.subcore_barrier()

        for b in range(NB):
            pltpu.sync_copy(col_hbm.at[wid * NB + b], colb)
            pend = [
                pltpu.async_copy(onesb, acc.at[colb.at[j]], ssem, add=True)
                for j in range(CH2)
            ]
            for s in pend:
                s.wait()
        plsc.subcore_barrier()
        pltpu.sync_copy(acc.at[pl.ds(sid * RPT, RPT)], zb)
        pltpu.sync_copy(zb, out_hbm.at[wid])

    pl.run_scoped(
        _scoped,
        pltpu.VMEM((CH2, K), jnp.int32),
        pltpu.VMEM((K, DW), jnp.float32),
        pltpu.VMEM((RPT, DW), jnp.float32),
        pltpu.SemaphoreType.DMA(()),
    )


def _deg_scs(col_hbm, out_hbm, acc):
    pass


_deg_call = pl.kernel(
    [_deg_tec, _deg_scs],
    out_type=jax.ShapeDtypeStruct((NW, RPT, DW), jnp.float32),
    mesh=[_mesh, _smesh],
    scratch_types=[
        pltpu.VMEM_SHARED((N_PAD, DW), jnp.float32),
    ],
)


def _conv_tec(xs_hbm, row_hbm, col_hbm, zeros_hbm, out_hbm, acc):
    cid = lax.axis_index("c")
    sid = lax.axis_index("s")
    wid = cid * NS + sid

    def _scoped(rowb, colb, gbuf0, gbuf1, zb, gsem, ssem):
        gbufs = (gbuf0, gbuf1)
        pltpu.sync_copy(row_hbm.at[wid * NB], rowb)
        pltpu.sync_copy(col_hbm.at[wid * NB], colb)
        pend_g = pltpu.async_copy(xs_hbm.at[rowb.at[0]], gbufs[0], gsem)
        pend_j = 0

        pltpu.sync_copy(zeros_hbm, zb)
        for p in range(RPT // ZR):
            pltpu.sync_copy(zb, acc.at[pl.ds(sid * RPT + p * ZR, ZR)])
        plsc.subcore_barrier()

        for b in range(NB):
            if b > 0:
                blk = wid * NB + b
                pltpu.sync_copy(row_hbm.at[blk], rowb)
                pltpu.sync_copy(col_hbm.at[blk], colb)
                pend_g = pltpu.async_copy(xs_hbm.at[rowb.at[0]], gbufs[0],
                                          gsem)
                pend_j = 0
            pend_s = [None, None]
            for j in range(1, CH2):
                buf = gbufs[j % 2]
                if pend_s[j % 2] is not None:
                    pend_s[j % 2].wait()
                    pend_s[j % 2] = None
                g = pltpu.async_copy(xs_hbm.at[rowb.at[j]], buf, gsem)
                pend_g.wait()
                pend_s[pend_j % 2] = pltpu.async_copy(
                    gbufs[pend_j % 2], acc.at[colb.at[pend_j]], ssem,
                    add=True)
                pend_g = g
                pend_j = j
            pend_g.wait()
            last_s = pltpu.async_copy(
                gbufs[pend_j % 2], acc.at[colb.at[pend_j]], ssem, add=True)
            for s in pend_s:
                if s is not None:
                    s.wait()
            last_s.wait()
        plsc.subcore_barrier()
        for p in range(RPT // ZR):
            pltpu.sync_copy(acc.at[pl.ds(sid * RPT + p * ZR, ZR)], zb)
            pltpu.sync_copy(zb, out_hbm.at[wid * (RPT // ZR) + p])

    pl.run_scoped(
        _scoped,
        pltpu.VMEM((CH2, K), jnp.int32),
        pltpu.VMEM((CH2, K), jnp.int32),
        pltpu.VMEM((K, D), jnp.float32),
        pltpu.VMEM((K, D), jnp.float32),
        pltpu.VMEM((ZR, D), jnp.float32),
        pltpu.SemaphoreType.DMA(()),
        pltpu.SemaphoreType.DMA(()),
    )


def _conv_scs(xs_hbm, row_hbm, col_hbm, zeros_hbm, out_hbm, acc):
    pass


_conv_call = pl.kernel(
    [_conv_tec, _conv_scs],
    out_type=jax.ShapeDtypeStruct((NW * (RPT // ZR), ZR, D), jnp.float32),
    mesh=[_mesh, _smesh],
    scratch_types=[
        pltpu.VMEM_SHARED((N_PAD, D), jnp.float32),
    ],
)


def _elu(v):
    return jnp.where(v > 0, v, jnp.exp(jnp.minimum(v, 0.0)) - 1.0)


def _a_body(x_ref, wcat_ref, bcat_ref, wi_ref, bi_ref, h_ref):
    xb = x_ref[...]
    s = jnp.dot(xb[:, :D_CAT], wcat_ref[...], preferred_element_type=jnp.float32,
                precision=_HI) + bcat_ref[...]
    s = jnp.maximum(s, 0.0)
    h = (jnp.dot(xb[:, D_CAT:], wi_ref[...][:D_CONT],
                 preferred_element_type=jnp.float32, precision=_HI)
         + jnp.dot(s, wi_ref[...][D_CONT:],
                   preferred_element_type=jnp.float32, precision=_HI)
         + bi_ref[...])
    h_ref[...] = jnp.maximum(h, 0.0)


_a_call = pl.pallas_call(
    _a_body,
    grid=(NG,),
    in_specs=[
        pl.BlockSpec((R, D), lambda i: (i, 0)),
        pl.BlockSpec((D_CAT, D_CAT), lambda i: (0, 0)),
        pl.BlockSpec((1, D_CAT), lambda i: (0, 0)),
        pl.BlockSpec((D, D), lambda i: (0, 0)),
        pl.BlockSpec((1, D), lambda i: (0, 0)),
    ],
    out_specs=pl.BlockSpec((R, D), lambda i: (i, 0)),
    out_shape=jax.ShapeDtypeStruct((N, D), jnp.float32),
)


def _b_body(h_ref, degp_ref, w1_ref, xs1_ref, dis_ref):
    degp = degp_ref[...]
    deg = (degp[0, :, 0] + degp[1, :, 0] + 1.0)[:, None]
    dis = lax.rsqrt(deg)
    xs1_ref[...] = jnp.dot(h_ref[...], w1_ref[...],
                           preferred_element_type=jnp.float32,
                           precision=_HI) * dis
    dis_ref[...] = dis


_b_call = pl.pallas_call(
    _b_body,
    grid=(NG,),
    in_specs=[
        pl.BlockSpec((R, D), lambda i: (i, 0)),
        pl.BlockSpec((NC, R, DW), lambda i: (0, i, 0)),
        pl.BlockSpec((D, D), lambda i: (0, 0)),
    ],
    out_specs=[
        pl.BlockSpec((R, D), lambda i: (i, 0)),
        pl.BlockSpec((R, 1), lambda i: (i, 0)),
    ],
    out_shape=[
        jax.ShapeDtypeStruct((N, D), jnp.float32),
        jax.ShapeDtypeStruct((N, 1), jnp.float32),
    ],
)


def _c_body(acc_ref, xs_ref, dis_ref, b_ref, w_ref, out_ref):
    a = acc_ref[0] + acc_ref[1] + xs_ref[...]
    dis = dis_ref[...]
    h = _elu(a * dis + b_ref[...])
    out_ref[...] = jnp.dot(h, w_ref[...], preferred_element_type=jnp.float32,
                           precision=_HI) * dis


_c_call = pl.pallas_call(
    _c_body,
    grid=(NG,),
    in_specs=[
        pl.BlockSpec((NC, R, D), lambda i: (0, i, 0)),
        pl.BlockSpec((R, D), lambda i: (i, 0)),
        pl.BlockSpec((R, 1), lambda i: (i, 0)),
        pl.BlockSpec((1, D), lambda i: (0, 0)),
        pl.BlockSpec((D, D), lambda i: (0, 0)),
    ],
    out_specs=pl.BlockSpec((R, D), lambda i: (i, 0)),
    out_shape=jax.ShapeDtypeStruct((N, D), jnp.float32),
)


def _d_body(acc_ref, xs_ref, dis_ref, b_ref, batch_ref, wout_ref, bout_ref,
            out_ref, pooled_acc, cnt_acc):
    i = pl.program_id(0)

    @pl.when(i == 0)
    def _():
        pooled_acc[...] = jnp.zeros_like(pooled_acc)
        cnt_acc[...] = jnp.zeros_like(cnt_acc)

    a = acc_ref[0] + acc_ref[1] + xs_ref[...]
    dis = dis_ref[...]
    h = _elu(a * dis + b_ref[...])
    bb = batch_ref[0, 0, :]
    oh = (bb[:, None] == lax.broadcasted_iota(jnp.int32, (R, G), 1)
          ).astype(jnp.float32)
    pooled_acc[...] += lax.dot_general(
        oh, h, (((0,), (0,)), ((), ())), preferred_element_type=jnp.float32,
        precision=_HI)
    cnt_acc[...] += jnp.sum(oh, axis=0)[:, None]

    @pl.when(i == pl.num_programs(0) - 1)
    def _():
        pooled = pooled_acc[...] / jnp.maximum(cnt_acc[...], 1.0)
        out_ref[...] = jnp.dot(pooled, wout_ref[...],
                               preferred_element_type=jnp.float32,
                               precision=_HI) + bout_ref[...]


_d_call = pl.pallas_call(
    _d_body,
    grid=(NG,),
    in_specs=[
        pl.BlockSpec((NC, R, D), lambda i: (0, i, 0)),
        pl.BlockSpec((R, D), lambda i: (i, 0)),
        pl.BlockSpec((R, 1), lambda i: (i, 0)),
        pl.BlockSpec((1, D), lambda i: (0, 0)),
        pl.BlockSpec((1, 1, R), lambda i: (i, 0, 0)),
        pl.BlockSpec((D, D), lambda i: (0, 0)),
        pl.BlockSpec((1, D), lambda i: (0, 0)),
    ],
    out_specs=pl.BlockSpec((G, D), lambda i: (0, 0)),
    out_shape=jax.ShapeDtypeStruct((G, D), jnp.float32),
    scratch_shapes=[
        pltpu.VMEM((G, D), jnp.float32),
        pltpu.VMEM((G, 1), jnp.float32),
    ],
    compiler_params=pltpu.CompilerParams(
        dimension_semantics=("arbitrary",),
    ),
)


def kernel(x, edge_attr, W_cat, b_cat, W_init, b_init, W1, b1, W2, b2,
           W_out, b_out, edge_index, batch):
    del edge_attr
    row = edge_index[0]
    col = edge_index[1]

    col2 = col.reshape(CHT // CH2, CH2, K)
    degp = _deg_call(col2).reshape(NC, N_PAD, DW)

    zrow = jnp.zeros((ZR, D), jnp.float32)
    row2 = row.reshape(CHT // CH2, CH2, K)

    h = _a_call(x, W_cat.T, b_cat[None, :], W_init.T, b_init[None, :])
    xs1, dis = _b_call(h, degp, W1.T)
    acc1 = _conv_call(xs1, row2, col2, zrow).reshape(NC, N_PAD, D)
    xs2 = _c_call(acc1, xs1, dis, b1[None, :], W2.T)
    acc2 = _conv_call(xs2, row2, col2, zrow).reshape(NC, N_PAD, D)
    out = _d_call(acc2, xs2, dis, b2[None, :], batch.reshape(NG, 1, R),
                  W_out.T, b_out[None, :])
    return out
